# baseline (speedup 1.0000x reference)
import functools

import jax
import jax.numpy as jnp
from jax import lax
from jax.experimental import pallas as pl
from jax.experimental.pallas import tpu as pltpu
from jax.experimental.pallas import tpu_sc as plsc

N_ROWS = 16384
ROW_W = 3200
D_IDX = 200
EMBED = 16

NC = 2
NS = 16
NW = NC * NS

ROWS_PER_W = N_ROWS // NW
BR = 4
N_BLK = ROWS_PER_W // BR
G_BLK = BR * D_IDX
CHUNK = 128
TAIL = D_IDX - CHUNK
NBUF = 4
PREF = 2


def _sc_body(x_hbm, p_hbm, tab_hbm, out_hbm, idx_v, x_v, pe_v, sem_ld,
             sem_st, sem_g):
    wid = lax.axis_index("s") * NC + lax.axis_index("c")
    r0w = wid * ROWS_PER_W

    def start_loads(blk_i, j):
        r0 = r0w + blk_i * BR
        pltpu.async_copy(p_hbm.at[pl.ds(r0, BR), :], idx_v.at[j],
                         sem_ld.at[j])
        pltpu.async_copy(x_hbm.at[pl.ds(r0, BR), :], x_v.at[j],
                         sem_ld.at[j])

    def wait_loads(blk_i, j):
        r0 = r0w + blk_i * BR
        pltpu.make_async_copy(p_hbm.at[pl.ds(r0, BR), :], idx_v.at[j],
                              sem_ld.at[j]).wait()
        pltpu.make_async_copy(x_hbm.at[pl.ds(r0, BR), :], x_v.at[j],
                              sem_ld.at[j]).wait()

    def start_store(blk_i, j):
        r0 = r0w + blk_i * BR
        pltpu.async_copy(x_v.at[j], out_hbm.at[pl.ds(r0, BR), :],
                         sem_st.at[j])

    def wait_store(blk_i, j):
        r0 = r0w + blk_i * BR
        pltpu.make_async_copy(x_v.at[j], out_hbm.at[pl.ds(r0, BR), :],
                              sem_st.at[j]).wait()

    def gather_block(j):
        copies = []
        for r in range(BR):
            copies.append(pltpu.async_copy(
                tab_hbm.at[idx_v.at[j, r, pl.ds(0, CHUNK)]],
                pe_v.at[pl.ds(r * D_IDX, CHUNK), :],
                sem_g,
            ))
            copies.append(pltpu.async_copy(
                tab_hbm.at[idx_v.at[j, r, pl.ds(CHUNK, TAIL)]],
                pe_v.at[pl.ds(r * D_IDX + CHUNK, TAIL), :],
                sem_g,
            ))
        for cp in copies:
            cp.wait()

    def add_block(j):
        for r in range(BR):
            def add_i(i, carry, r=r):
                plsc.addupdate(
                    x_v.at[j, r, pl.ds(pl.multiple_of(i * EMBED, EMBED),
                                       EMBED)],
                    pe_v[r * D_IDX + i, :],
                )
                return carry

            lax.fori_loop(0, D_IDX, add_i, 0, unroll=8)

    for j in range(PREF):
        start_loads(j, j)

    @pl.loop(0, N_BLK, step=NBUF)
    def blk_loop(b0):
        for j in range(NBUF):
            b = b0 + j
            nb = b + PREF
            jn = (j + PREF) % NBUF

            @pl.when(nb < N_BLK)
            def _prefetch():
                @pl.when(b >= PREF)
                def _drain_store():
                    wait_store(b - PREF, jn)
                start_loads(nb, jn)

            wait_loads(b, j)
            gather_block(j)
            add_block(j)
            start_store(b, j)

    for b in range(N_BLK - NBUF, N_BLK):
        wait_store(b, b % NBUF)


@functools.partial(jax.jit, static_argnames=())
def _run(x, p, tab):
    mesh = plsc.VectorSubcoreMesh(
        core_axis_name="c", subcore_axis_name="s", num_cores=NC,
        num_subcores=NS,
    )
    return pl.kernel(
        _sc_body,
        out_type=jax.ShapeDtypeStruct((N_ROWS, ROW_W), jnp.float32),
        mesh=mesh,
        scratch_types=[
            pltpu.VMEM((NBUF, BR, D_IDX), jnp.int32),
            pltpu.VMEM((NBUF, BR, ROW_W), jnp.float32),
            pltpu.VMEM((G_BLK, EMBED), jnp.float32),
            pltpu.SemaphoreType.DMA((NBUF,)),
            pltpu.SemaphoreType.DMA((NBUF,)),
            pltpu.SemaphoreType.DMA,
        ],
        compiler_params=pltpu.CompilerParams(use_tc_tiling_on_sc=False),
    )(x, p, tab)


def kernel(x, p, pos_table):
    return _run(x, p.astype(jnp.int32), pos_table)


# speedup vs baseline: 1.3619x; 1.3619x over previous
import functools

import jax
import jax.numpy as jnp
from jax import lax
from jax.experimental import pallas as pl
from jax.experimental.pallas import tpu as pltpu
from jax.experimental.pallas import tpu_sc as plsc

N_ROWS = 16384
ROW_W = 3200
D_IDX = 200
EMBED = 16

NC = 2
NS = 16
NW = NC * NS

N_CHUNK = 2
CH_ROWS = N_ROWS // N_CHUNK
N_G = CH_ROWS * D_IDX
G_PER_W = N_G // NW
BLK = 1600
N_BLK = G_PER_W // BLK
CHUNK = 128
N_FULL = BLK // CHUNK
TAIL = BLK - N_FULL * CHUNK
NBUF = 4
PREF = 2


def _sc_body(p_hbm, tab_hbm, pe_hbm, idx_v, pe_v, sem_ld, sem_st, sem_g):
    wid = lax.axis_index("s") * NC + lax.axis_index("c")
    g0 = wid * G_PER_W

    def start_load(blk_i, j):
        base = g0 + blk_i * BLK
        pltpu.async_copy(p_hbm.at[pl.ds(base, BLK)], idx_v.at[j],
                         sem_ld.at[j])

    def wait_load(blk_i, j):
        base = g0 + blk_i * BLK
        pltpu.make_async_copy(p_hbm.at[pl.ds(base, BLK)], idx_v.at[j],
                              sem_ld.at[j]).wait()

    def start_store(blk_i, j):
        base = g0 + blk_i * BLK
        pltpu.async_copy(pe_v.at[j], pe_hbm.at[pl.ds(base, BLK), :],
                         sem_st.at[j])

    def wait_store(blk_i, j):
        base = g0 + blk_i * BLK
        pltpu.make_async_copy(pe_v.at[j], pe_hbm.at[pl.ds(base, BLK), :],
                              sem_st.at[j]).wait()

    def gather_block(j):
        copies = []
        for c in range(N_FULL):
            copies.append(pltpu.async_copy(
                tab_hbm.at[idx_v.at[j, pl.ds(c * CHUNK, CHUNK)]],
                pe_v.at[j, pl.ds(c * CHUNK, CHUNK), :],
                sem_g,
            ))
        copies.append(pltpu.async_copy(
            tab_hbm.at[idx_v.at[j, pl.ds(N_FULL * CHUNK, TAIL)]],
            pe_v.at[j, pl.ds(N_FULL * CHUNK, TAIL), :],
            sem_g,
        ))
        for cp in copies:
            cp.wait()

    for j in range(PREF):
        start_load(j, j)

    @pl.loop(0, N_BLK, step=NBUF)
    def blk_loop(b0):
        for j in range(NBUF):
            b = b0 + j
            nb = b + PREF
            jn = (j + PREF) % NBUF

            @pl.when(nb < N_BLK)
            def _prefetch():
                @pl.when(b >= PREF)
                def _drain_store():
                    wait_store(b - PREF, jn)
                start_load(nb, jn)

            wait_load(b, j)
            gather_block(j)
            start_store(b, j)

    for b in range(N_BLK - NBUF, N_BLK):
        wait_store(b, b % NBUF)


def _sc_gather(pf, tab):
    mesh = plsc.VectorSubcoreMesh(
        core_axis_name="c", subcore_axis_name="s", num_cores=NC,
        num_subcores=NS,
    )
    return pl.kernel(
        _sc_body,
        out_type=jax.ShapeDtypeStruct((N_G, EMBED), jnp.float32),
        mesh=mesh,
        scratch_types=[
            pltpu.VMEM((NBUF, BLK), jnp.int32),
            pltpu.VMEM((NBUF, BLK, EMBED), jnp.float32),
            pltpu.SemaphoreType.DMA((NBUF,)),
            pltpu.SemaphoreType.DMA((NBUF,)),
            pltpu.SemaphoreType.DMA,
        ],
        compiler_params=pltpu.CompilerParams(use_tc_tiling_on_sc=False),
    )(pf, tab)


@functools.partial(jax.jit, static_argnames=())
def _run(x, p, tab):
    outs = []
    for k in range(N_CHUNK):
        pk = p[k * CH_ROWS:(k + 1) * CH_ROWS].reshape(N_G)
        pe = _sc_gather(pk, tab)
        xk = x[k * CH_ROWS:(k + 1) * CH_ROWS]
        outs.append(xk + pe.reshape(CH_ROWS, ROW_W))
    return jnp.concatenate(outs, axis=0)


def kernel(x, p, pos_table):
    return _run(x, p.astype(jnp.int32), pos_table)
